# baseline (speedup 1.0000x reference)
import functools
import jax
import jax.numpy as jnp
from jax import lax
from jax.experimental import pallas as pl
from jax.experimental.pallas import tpu as pltpu

N = 10000
E = 320000
D = 128
H = 128


def _k1_body(x_ref, wl_ref, bl_ref, w1_ref, p_ref, q_ref):
    h = jnp.dot(x_ref[...], wl_ref[...], preferred_element_type=jnp.float32)
    h = h + bl_ref[...]
    wa = w1_ref[:D, :] - w1_ref[D:, :]
    wb = w1_ref[D:, :]
    p_ref[...] = jnp.dot(h, wa, preferred_element_type=jnp.float32)
    q_ref[...] = jnp.dot(h, wb, preferred_element_type=jnp.float32)


def _node_matmuls(x, W_lin1, b_lin1, W1):
    blk = 1000
    grid = (N // blk,)
    return pl.pallas_call(
        _k1_body,
        grid=grid,
        in_specs=[
            pl.BlockSpec((blk, D), lambda i: (i, 0)),
            pl.BlockSpec((D, D), lambda i: (0, 0)),
            pl.BlockSpec((1, D), lambda i: (0, 0)),
            pl.BlockSpec((2 * D, H), lambda i: (0, 0)),
        ],
        out_specs=[
            pl.BlockSpec((blk, H), lambda i: (i, 0)),
            pl.BlockSpec((blk, H), lambda i: (i, 0)),
        ],
        out_shape=[
            jax.ShapeDtypeStruct((N, H), jnp.float32),
            jax.ShapeDtypeStruct((N, H), jnp.float32),
        ],
    )(x, W_lin1, b_lin1.reshape(1, D), W1)


def _k3_body(zp_ref, zq_ref, b1_ref, w2_ref, b2_ref, y_ref):
    z = jnp.maximum(zp_ref[...] + zq_ref[...] + b1_ref[...], 0.0)
    y = jnp.dot(z, w2_ref[...], preferred_element_type=jnp.float32)
    y_ref[...] = jnp.maximum(y + b2_ref[...], 0.0)


def _edge_mlp(zp, zq, b1, W2, b2):
    blk = 2000
    grid = (E // blk,)
    return pl.pallas_call(
        _k3_body,
        grid=grid,
        in_specs=[
            pl.BlockSpec((blk, H), lambda i: (i, 0)),
            pl.BlockSpec((blk, H), lambda i: (i, 0)),
            pl.BlockSpec((1, H), lambda i: (0, 0)),
            pl.BlockSpec((H, H), lambda i: (0, 0)),
            pl.BlockSpec((1, H), lambda i: (0, 0)),
        ],
        out_specs=pl.BlockSpec((blk, H), lambda i: (i, 0)),
        out_shape=jax.ShapeDtypeStruct((E, H), jnp.float32),
    )(zp, zq, b1.reshape(1, H), W2, b2.reshape(1, H))


def kernel(x, edge_index, W_lin1, b_lin1, W1, b1, W2, b2):
    src = edge_index[0]
    dst = edge_index[1]
    p, q = _node_matmuls(x, W_lin1, b_lin1, W1)
    zp = jnp.take(p, dst, axis=0)
    zq = jnp.take(q, src, axis=0)
    y = _edge_mlp(zp, zq, b1, W2, b2)
    agg = jax.ops.segment_max(y, dst, num_segments=N)
    agg = jnp.where(jnp.isfinite(agg), agg, 0.0)
    return jnp.maximum(agg, 0.0)


# speedup vs baseline: 1.7092x; 1.7092x over previous
import functools
import jax
import jax.numpy as jnp
from jax import lax
from jax.experimental import pallas as pl
from jax.experimental.pallas import tpu as pltpu
from jax.experimental.pallas import tpu_sc as plsc

N = 10000
E = 320000
D = 128
H = 128

_NC = 2
_NS = 16
_NW = _NC * _NS


def _k1_body(x_ref, wl_ref, bl_ref, w1_ref, p_ref, q_ref):
    h = jnp.dot(x_ref[...], wl_ref[...], preferred_element_type=jnp.float32)
    h = h + bl_ref[...]
    wa = w1_ref[:D, :] - w1_ref[D:, :]
    wb = w1_ref[D:, :]
    p_ref[...] = jnp.dot(h, wa, preferred_element_type=jnp.float32)
    q_ref[...] = jnp.dot(h, wb, preferred_element_type=jnp.float32)


def _node_matmuls(x, W_lin1, b_lin1, W1):
    blk = 1000
    grid = (N // blk,)
    return pl.pallas_call(
        _k1_body,
        grid=grid,
        in_specs=[
            pl.BlockSpec((blk, D), lambda i: (i, 0)),
            pl.BlockSpec((D, D), lambda i: (0, 0)),
            pl.BlockSpec((1, D), lambda i: (0, 0)),
            pl.BlockSpec((2 * D, H), lambda i: (0, 0)),
        ],
        out_specs=[
            pl.BlockSpec((blk, H), lambda i: (i, 0)),
            pl.BlockSpec((blk, H), lambda i: (i, 0)),
        ],
        out_shape=[
            jax.ShapeDtypeStruct((N, H), jnp.float32),
            jax.ShapeDtypeStruct((N, H), jnp.float32),
        ],
    )(x, W_lin1, b_lin1.reshape(1, D), W1)


_EPW = E // _NW
_CH = 400


def _k2_body(dst_hbm, src_hbm, p_hbm, q_hbm, z_hbm,
             idxd_v, idxs_v, bufp, bufq, semp, semq):
    wid = lax.axis_index("s") * _NC + lax.axis_index("c")
    base = wid * _EPW

    def chunk(i, carry):
        off = base + i * _CH
        pltpu.sync_copy(dst_hbm.at[pl.ds(off, _CH)], idxd_v)
        pltpu.sync_copy(src_hbm.at[pl.ds(off, _CH)], idxs_v)
        cp = pltpu.async_copy(p_hbm.at[idxd_v], bufp, semp)
        cq = pltpu.async_copy(q_hbm.at[idxs_v], bufq, semq)
        cp.wait()
        cq.wait()

        def row(r, c2):
            for c in range(H // 16):
                s = pl.ds(c * 16, 16)
                bufp[r, s] = bufp[r, s] + bufq[r, s]
            return c2

        lax.fori_loop(0, _CH, row, 0)
        pltpu.sync_copy(bufp, z_hbm.at[pl.ds(off, _CH)])
        return carry

    lax.fori_loop(0, _EPW // _CH, chunk, 0)


def _edge_gather(dst, src, p, q):
    mesh = plsc.VectorSubcoreMesh(core_axis_name="c", subcore_axis_name="s")
    f = functools.partial(
        pl.kernel,
        out_type=jax.ShapeDtypeStruct((E, H), jnp.float32),
        mesh=mesh,
        compiler_params=pltpu.CompilerParams(needs_layout_passes=False),
        scratch_types=[
            pltpu.VMEM((_CH,), jnp.int32),
            pltpu.VMEM((_CH,), jnp.int32),
            pltpu.VMEM((_CH, H), jnp.float32),
            pltpu.VMEM((_CH, H), jnp.float32),
            pltpu.SemaphoreType.DMA,
            pltpu.SemaphoreType.DMA,
        ],
    )(_k2_body)
    return f(dst, src, p, q)


def _k3_body(z_ref, b1_ref, w2_ref, b2_ref, y_ref):
    z = jnp.maximum(z_ref[...] + b1_ref[...], 0.0)
    y = jnp.dot(z, w2_ref[...], preferred_element_type=jnp.float32)
    y_ref[...] = jnp.maximum(y + b2_ref[...], 0.0)


def _edge_mlp(z, b1, W2, b2):
    blk = 2000
    grid = (E // blk,)
    return pl.pallas_call(
        _k3_body,
        grid=grid,
        in_specs=[
            pl.BlockSpec((blk, H), lambda i: (i, 0)),
            pl.BlockSpec((1, H), lambda i: (0, 0)),
            pl.BlockSpec((H, H), lambda i: (0, 0)),
            pl.BlockSpec((1, H), lambda i: (0, 0)),
        ],
        out_specs=pl.BlockSpec((blk, H), lambda i: (i, 0)),
        out_shape=jax.ShapeDtypeStruct((E, H), jnp.float32),
    )(z, b1.reshape(1, H), W2, b2.reshape(1, H))


_G = 320
_NPAD = _NW * _G
_CH2 = 2000
_GCH = 256


def _k4_body(dst_hbm, y_hbm, out_hbm, dwin, pid, pd, rows, acc, semg):
    wid = lax.axis_index("s") * _NC + lax.axis_index("c")
    lo = wid * _G
    iota = lax.iota(jnp.int32, 16)
    zeros16 = jnp.zeros((16,), jnp.float32)

    def zrow(r, c2):
        for c in range(H // 16):
            acc[r, pl.ds(c * 16, 16)] = zeros16
        return c2

    lax.fori_loop(0, _G, zrow, 0)

    def zpid(i, c2):
        pid[pl.ds(i * 16, 16)] = jnp.zeros((16,), jnp.int32)
        return c2

    lax.fori_loop(0, _GCH // 16, zpid, 0)

    def flush(cnt):
        pltpu.async_copy(y_hbm.at[pid], rows, semg).wait()

        def rmw(j, c2):
            jvec = jnp.full((16,), j, jnp.int32)
            dvec = plsc.load_gather(pd, [jvec]) - lo
            for c in range(H // 16):
                colv = jnp.full((16,), c * 16, jnp.int32) + iota
                cur = plsc.load_gather(acc, [dvec, colv])
                yv = rows[j, pl.ds(c * 16, 16)]
                plsc.store_scatter(acc, [dvec, colv], jnp.maximum(cur, yv))
            return c2

        lax.fori_loop(0, cnt, rmw, 0)

    def chunk(ci, off):
        cb = ci * _CH2
        pltpu.sync_copy(dst_hbm.at[pl.ds(cb, _CH2)], dwin)

        def vb(v, off):
            d = dwin[pl.ds(v * 16, 16)]
            mask = (d >= lo) & (d < lo + _G)
            mi = jnp.where(mask, 1, 0)
            pos = plsc.cumsum(mi) - 1 + off
            eid = cb + v * 16 + iota
            plsc.store_scatter(pid, [pos], eid, mask=mask)
            plsc.store_scatter(pd, [pos], d, mask=mask)
            off = off + plsc.all_reduce_population_count(mask)[0]

            def do_flush(o):
                flush(o)
                return jnp.int32(0)

            return lax.cond(off >= _GCH - 16, do_flush, lambda o: o, off)

        return lax.fori_loop(0, _CH2 // 16, vb, off)

    off = lax.fori_loop(0, E // _CH2, chunk, jnp.int32(0))
    flush(off)
    pltpu.sync_copy(acc, out_hbm.at[pl.ds(lo, _G)])


def _scatter_max(dst, y):
    mesh = plsc.VectorSubcoreMesh(core_axis_name="c", subcore_axis_name="s")
    f = functools.partial(
        pl.kernel,
        out_type=jax.ShapeDtypeStruct((_NPAD, H), jnp.float32),
        mesh=mesh,
        compiler_params=pltpu.CompilerParams(needs_layout_passes=False),
        scratch_types=[
            pltpu.VMEM((_CH2,), jnp.int32),
            pltpu.VMEM((_GCH,), jnp.int32),
            pltpu.VMEM((_GCH,), jnp.int32),
            pltpu.VMEM((_GCH, H), jnp.float32),
            pltpu.VMEM((_G, H), jnp.float32),
            pltpu.SemaphoreType.DMA,
        ],
    )(_k4_body)
    return f(dst, y)


def kernel(x, edge_index, W_lin1, b_lin1, W1, b1, W2, b2):
    src = edge_index[0]
    dst = edge_index[1]
    p, q = _node_matmuls(x, W_lin1, b_lin1, W1)
    z = _edge_gather(dst, src, p, q)
    y = _edge_mlp(z, b1, W2, b2)
    agg = _scatter_max(dst, y)
    return agg[:N]
